# baseline (speedup 1.0000x reference)
import functools

import jax
import jax.numpy as jnp
from jax import lax
from jax.experimental import pallas as pl
from jax.experimental.pallas import tpu as pltpu
from jax.experimental.pallas import tpu_sc as plsc

N_CLASSES = 1000
N_ROWS = 16384
SMOOTH_ = 0.1
FILL_VAL = SMOOTH_ / N_CLASSES
HOT_VAL = 1.0 - SMOOTH_ + SMOOTH_ / N_CLASSES

NUM_CORES = 2
NUM_SUBCORES = 16
LANES = 16
NUM_WORKERS = NUM_CORES * NUM_SUBCORES
ROWS_PER_WORKER = N_ROWS // NUM_WORKERS
R = 16
ITERS = ROWS_PER_WORKER // R
CHUNK = R * N_CLASSES


def _sc_body(target_hbm, out_hbm, idx_v, buf0, buf1, sem0, sem1):
    wid = lax.axis_index("s") * NUM_CORES + lax.axis_index("c")
    base_row = wid * ROWS_PER_WORKER
    pltpu.sync_copy(target_hbm.at[pl.ds(base_row, ROWS_PER_WORKER)], idx_v)

    cfill = jnp.full((LANES,), FILL_VAL, jnp.float32)
    chot = jnp.full((LANES,), HOT_VAL, jnp.float32)

    def fill(k, carry):
        buf0[pl.ds(k * LANES, LANES)] = cfill
        buf1[pl.ds(k * LANES, LANES)] = cfill
        return carry

    lax.fori_loop(0, CHUNK // LANES, fill, 0)

    row_off = lax.iota(jnp.int32, LANES) * jnp.int32(N_CLASSES)

    bufs = (buf0, buf1)
    sems = (sem0, sem1)
    handles = [None, None]
    pending_pos = [None, None]
    for i in range(ITERS):
        b = i % 2
        buf = bufs[b]
        if handles[b] is not None:
            handles[b].wait()
            plsc.store_scatter(buf, [pending_pos[b]], cfill)
        pos = row_off + idx_v[pl.ds(i * R, LANES)]
        plsc.store_scatter(buf, [pos], chot)
        pending_pos[b] = pos
        handles[b] = pltpu.async_copy(
            buf,
            out_hbm.at[pl.ds((base_row + i * R) * N_CLASSES, CHUNK)],
            sems[b],
        )
    handles[0].wait()
    handles[1].wait()


_sc_onehot = functools.partial(
    pl.kernel,
    out_type=jax.ShapeDtypeStruct((N_ROWS * N_CLASSES,), jnp.float32),
    scratch_types=[
        pltpu.VMEM((ROWS_PER_WORKER,), jnp.int32),
        pltpu.VMEM((CHUNK,), jnp.float32),
        pltpu.VMEM((CHUNK,), jnp.float32),
        pltpu.SemaphoreType.DMA,
        pltpu.SemaphoreType.DMA,
    ],
    mesh=plsc.VectorSubcoreMesh(core_axis_name="c", subcore_axis_name="s"),
    compiler_params=pltpu.CompilerParams(needs_layout_passes=False),
)(_sc_body)


def kernel(target):
    flat = _sc_onehot(target.astype(jnp.int32))
    return flat.reshape(N_ROWS, N_CLASSES)
